# baseline (speedup 1.0000x reference)
import functools

import jax
import jax.numpy as jnp
from jax import lax
from jax.experimental import pallas as pl
from jax.experimental.pallas import tpu as pltpu
from jax.experimental.pallas import tpu_sc as plsc

NN = 10000
EE = 320000
HH = 32
DD = 128

NC = 2
NS = 16
NW = NC * NS
EPW = EE // NW
CH = 128
EPWP = 10240
NCH = EPWP // CH
EPAD = EPWP - EPW
NP = NN + 8
ZR = 624
ZTAIL = NP - NS * ZR

NB = 5
PG = 3

_MESH = plsc.VectorSubcoreMesh(core_axis_name="c", subcore_axis_name="s")
_DIMS = (((1,), (1,)), ((), ()))


def _sc_agg_common(p_hbm, srcr, dstr, zeros32, out_hbm,
                   acc, src_all, dst_all, rows, gsem, ssem,
                   cnt=None):
    c = lax.axis_index("c")
    s = lax.axis_index("s")
    wid = c * NS + s
    if cnt is not None:
        zeros16, ones16, cnt_hbm, acc16, ones_v, osem = cnt

    def fire_gather(j, b):
        pltpu.async_copy(p_hbm.at[src_all.at[j]], rows.at[b], gsem.at[b])

    def wait_gather(j, b):
        pltpu.make_async_copy(p_hbm.at[src_all.at[j]], rows.at[b],
                              gsem.at[b]).wait()

    def fire_scatter(j, b):
        pltpu.async_copy(rows.at[b], acc.at[dst_all.at[j]], ssem.at[b],
                         add=True)
        if cnt is not None:
            pltpu.async_copy(ones_v, acc16.at[dst_all.at[j]], osem.at[b],
                             add=True)

    def wait_scatter(j, b):
        pltpu.make_async_copy(rows.at[b], acc.at[dst_all.at[j]],
                              ssem.at[b]).wait()
        if cnt is not None:
            pltpu.make_async_copy(ones_v, acc16.at[dst_all.at[j]],
                                  osem.at[b]).wait()

    pltpu.sync_copy(srcr.at[wid], src_all)
    pltpu.sync_copy(dstr.at[wid], dst_all)
    pltpu.sync_copy(zeros32.at[pl.ds(s * ZR, ZR)], acc.at[pl.ds(s * ZR, ZR)])
    if cnt is not None:
        pltpu.sync_copy(zeros16.at[pl.ds(s * ZR, ZR)],
                        acc16.at[pl.ds(s * ZR, ZR)])
        pltpu.sync_copy(ones16, ones_v)

    @pl.when(s == NS - 1)
    def _():
        pltpu.sync_copy(zeros32.at[pl.ds(NS * ZR, ZTAIL)],
                        acc.at[pl.ds(NS * ZR, ZTAIL)])
        if cnt is not None:
            pltpu.sync_copy(zeros16.at[pl.ds(NS * ZR, ZTAIL)],
                            acc16.at[pl.ds(NS * ZR, ZTAIL)])

    plsc.subcore_barrier()

    for j in range(PG):
        fire_gather(j, j % NB)

    for b in range(NB):
        jg = b + PG
        if b >= NB - PG:
            wait_scatter(b - (NB - PG), jg % NB)
        fire_gather(jg, jg % NB)
        wait_gather(b, b)
        fire_scatter(b, b)

    @pl.loop(NB, NCH - NB, step=NB)
    def _(j0):
        for b in range(NB):
            j = j0 + b
            jg = j + PG
            bg = (b + PG) % NB
            wait_scatter(jg - NB, bg)
            fire_gather(jg, bg)
            wait_gather(j, b)
            fire_scatter(j, b)

    for b in range(NB):
        j = NCH - NB + b
        jg = j + PG
        if jg < NCH:
            wait_scatter(jg - NB, jg % NB)
            fire_gather(jg, jg % NB)
        wait_gather(j, b)
        fire_scatter(j, b)
    for b in range(NB):
        wait_scatter(NCH - NB + b, b)

    plsc.subcore_barrier()
    pltpu.sync_copy(acc.at[pl.ds(s * ZR, ZR)],
                    out_hbm.at[c, pl.ds(s * ZR, ZR)])
    if cnt is not None:
        pltpu.sync_copy(acc16.at[pl.ds(s * ZR, ZR)],
                        cnt_hbm.at[c, pl.ds(s * ZR, ZR)])

    @pl.when(s == NS - 1)
    def _():
        pltpu.sync_copy(acc.at[pl.ds(NS * ZR, ZTAIL)],
                        out_hbm.at[c, pl.ds(NS * ZR, ZTAIL)])
        if cnt is not None:
            pltpu.sync_copy(acc16.at[pl.ds(NS * ZR, ZTAIL)],
                            cnt_hbm.at[c, pl.ds(NS * ZR, ZTAIL)])


def _sc_first_body(p_hbm, srcr, dstr, zeros32, zeros16, ones16,
                   out_hbm, cnt_hbm,
                   acc, acc16, src_all, dst_all, rows, ones_v,
                   gsem, ssem, osem):
    _sc_agg_common(p_hbm, srcr, dstr, zeros32, out_hbm,
                   acc, src_all, dst_all, rows, gsem, ssem,
                   cnt=(zeros16, ones16, cnt_hbm, acc16, ones_v, osem))


def _sc_rest_body(p_hbm, srcr, dstr, zeros32, out_hbm,
                  acc, src_all, dst_all, rows, gsem, ssem):
    _sc_agg_common(p_hbm, srcr, dstr, zeros32, out_hbm,
                   acc, src_all, dst_all, rows, gsem, ssem)


_sc_agg_first = pl.kernel(
    _sc_first_body,
    out_type=(jax.ShapeDtypeStruct((NC, NP, HH), jnp.float32),
              jax.ShapeDtypeStruct((NC, NP, 16), jnp.float32)),
    mesh=_MESH,
    compiler_params=pltpu.CompilerParams(use_tc_tiling_on_sc=False),
    scratch_types=[
        pltpu.VMEM_SHARED((NP, HH), jnp.float32),
        pltpu.VMEM_SHARED((NP, 16), jnp.float32),
        pltpu.VMEM((NCH, CH), jnp.int32),
        pltpu.VMEM((NCH, CH), jnp.int32),
        pltpu.VMEM((NB, CH, HH), jnp.float32),
        pltpu.VMEM((CH, 16), jnp.float32),
        pltpu.SemaphoreType.DMA((NB,)),
        pltpu.SemaphoreType.DMA((NB,)),
        pltpu.SemaphoreType.DMA((NB,)),
    ],
)

_sc_agg_rest = pl.kernel(
    _sc_rest_body,
    out_type=jax.ShapeDtypeStruct((NC, NP, HH), jnp.float32),
    mesh=_MESH,
    compiler_params=pltpu.CompilerParams(use_tc_tiling_on_sc=False),
    scratch_types=[
        pltpu.VMEM_SHARED((NP, HH), jnp.float32),
        pltpu.VMEM((NCH, CH), jnp.int32),
        pltpu.VMEM((NCH, CH), jnp.int32),
        pltpu.VMEM((NB, CH, HH), jnp.float32),
        pltpu.SemaphoreType.DMA((NB,)),
        pltpu.SemaphoreType.DMA((NB,)),
    ],
)


def _proj0_body(x_ref, a_ref, b_ref, p_ref, q_ref):
    x = x_ref[...]
    p_ref[...] = lax.dot_general(x, a_ref[...], _DIMS,
                                 preferred_element_type=jnp.float32)
    q_ref[...] = lax.dot_general(x, b_ref[...], _DIMS,
                                 preferred_element_type=jnp.float32)


def _softmax1(v):
    m = jnp.max(v, axis=1, keepdims=True)
    e = jnp.exp(v - m)
    return e / jnp.sum(e, axis=1, keepdims=True)


def _softmax0(v):
    m = jnp.max(v, axis=0, keepdims=True)
    e = jnp.exp(v - m)
    return e / jnp.sum(e, axis=0, keepdims=True)


def _combine(aggp_ref, q_ref, invc_ref):
    agg = aggp_ref[0, 0:NN, :] + aggp_ref[1, 0:NN, :]
    return agg * invc_ref[...] + q_ref[...]


def _round_body(aggp_ref, q_ref, invc_ref, a_ref, b_ref, p_out, q_out, *,
                do_sm0):
    h = _softmax1(_combine(aggp_ref, q_ref, invc_ref))
    if do_sm0:
        h = _softmax0(h)
    p_out[...] = lax.dot_general(h, a_ref[...], _DIMS,
                                 preferred_element_type=jnp.float32)
    q_out[...] = lax.dot_general(h, b_ref[...], _DIMS,
                                 preferred_element_type=jnp.float32)


def _round0_body(aggp_ref, q_ref, cntp_ref, a_ref, b_ref,
                 p_out, q_out, invc_out):
    cnt = cntp_ref[0, 0:NN, 0:1] + cntp_ref[1, 0:NN, 0:1]
    invc_out[...] = jnp.broadcast_to(1.0 / jnp.maximum(cnt, 1.0), (NN, HH))
    h = _softmax1(_combine(aggp_ref, q_ref, invc_out))
    p_out[...] = lax.dot_general(h, a_ref[...], _DIMS,
                                 preferred_element_type=jnp.float32)
    q_out[...] = lax.dot_general(h, b_ref[...], _DIMS,
                                 preferred_element_type=jnp.float32)


def _final_body(aggp_ref, q_ref, invc_ref, w_ref, bo_ref, y_ref):
    h = _softmax0(_softmax1(_combine(aggp_ref, q_ref, invc_ref)))
    z = jnp.sum(h * w_ref[...], axis=1, keepdims=True) + bo_ref[0, 0]
    y_ref[...] = _softmax0(z)


_NH = jax.ShapeDtypeStruct((NN, HH), jnp.float32)

_proj0 = pl.pallas_call(
    _proj0_body, out_shape=(_NH, _NH))

_round0 = pl.pallas_call(
    _round0_body, out_shape=(_NH, _NH, _NH))
_round_sm = pl.pallas_call(
    functools.partial(_round_body, do_sm0=True), out_shape=(_NH, _NH))

_final = pl.pallas_call(
    _final_body, out_shape=jax.ShapeDtypeStruct((NN, 1), jnp.float32))


def kernel(x, edge_index, A0, B0, As, Bs, Wout, bout):
    src2 = edge_index[0].reshape(NW, EPW)
    dst2 = edge_index[1].reshape(NW, EPW)
    srcr = jnp.concatenate(
        [src2, jnp.zeros((NW, EPAD), jnp.int32)], axis=1).reshape(NW, NCH, CH)
    dstr = jnp.concatenate(
        [dst2, jnp.full((NW, EPAD), NN, jnp.int32)], axis=1).reshape(
            NW, NCH, CH)
    z32 = jnp.zeros((NP, HH), jnp.float32)
    z16 = jnp.zeros((NP, 16), jnp.float32)
    o16 = jnp.ones((CH, 16), jnp.float32)

    p, q = _proj0(x, A0, B0)
    aggp, cntp = _sc_agg_first(p, srcr, dstr, z32, z16, o16)
    p, q, invc = _round0(aggp, q, cntp, As[0], Bs[0])

    for r in range(1, 16):
        aggp = _sc_agg_rest(p, srcr, dstr, z32)
        p, q = _round_sm(aggp, q, invc, As[r], Bs[r])

    aggp = _sc_agg_rest(p, srcr, dstr, z32)
    return _final(aggp, q, invc, Wout, bout.reshape(1, 1))


# speedup vs baseline: 1.9740x; 1.9740x over previous
import functools

import jax
import jax.numpy as jnp
from jax import lax
from jax.experimental import pallas as pl
from jax.experimental.pallas import tpu as pltpu
from jax.experimental.pallas import tpu_sc as plsc

NN = 10000
EE = 320000
HH = 32
DD = 128

NC = 2
NS = 16
NW = NC * NS
EPW = EE // NW
CH = 80
EPWP = 10000
NCH = EPWP // CH
EPAD = EPWP - EPW
NP = NN + 8
ZR = 624
ZTAIL = NP - NS * ZR

NB = 5
PG = 3

_MESH = plsc.VectorSubcoreMesh(core_axis_name="c", subcore_axis_name="s")
_DIMS = (((1,), (1,)), ((), ()))


def _sc_agg_common(p_hbm, srcr, dstr, zeros32, out_hbm,
                   acc, src_all, dst_all, rows, gsem, ssem,
                   cnt=None):
    c = lax.axis_index("c")
    s = lax.axis_index("s")
    wid = c * NS + s
    if cnt is not None:
        zeros16, ones16, cnt_hbm, acc16, ones_v, osem = cnt

    def fire_gather(j, b):
        pltpu.async_copy(p_hbm.at[src_all.at[j]], rows.at[b], gsem.at[b])

    def wait_gather(j, b):
        pltpu.make_async_copy(p_hbm.at[src_all.at[j]], rows.at[b],
                              gsem.at[b]).wait()

    def fire_scatter(j, b):
        pltpu.async_copy(rows.at[b], acc.at[dst_all.at[j]], ssem.at[b],
                         add=True)
        if cnt is not None:
            pltpu.async_copy(ones_v, acc16.at[dst_all.at[j]], osem.at[b],
                             add=True)

    def wait_scatter(j, b):
        pltpu.make_async_copy(rows.at[b], acc.at[dst_all.at[j]],
                              ssem.at[b]).wait()
        if cnt is not None:
            pltpu.make_async_copy(ones_v, acc16.at[dst_all.at[j]],
                                  osem.at[b]).wait()

    pltpu.sync_copy(srcr.at[wid], src_all)
    pltpu.sync_copy(dstr.at[wid], dst_all)
    pltpu.sync_copy(zeros32.at[pl.ds(s * ZR, ZR)], acc.at[pl.ds(s * ZR, ZR)])
    if cnt is not None:
        pltpu.sync_copy(zeros16.at[pl.ds(s * ZR, ZR)],
                        acc16.at[pl.ds(s * ZR, ZR)])
        pltpu.sync_copy(ones16, ones_v)

    @pl.when(s == NS - 1)
    def _():
        pltpu.sync_copy(zeros32.at[pl.ds(NS * ZR, ZTAIL)],
                        acc.at[pl.ds(NS * ZR, ZTAIL)])
        if cnt is not None:
            pltpu.sync_copy(zeros16.at[pl.ds(NS * ZR, ZTAIL)],
                            acc16.at[pl.ds(NS * ZR, ZTAIL)])

    plsc.subcore_barrier()

    for j in range(PG):
        fire_gather(j, j % NB)

    for b in range(NB):
        jg = b + PG
        if b >= NB - PG:
            wait_scatter(b - (NB - PG), jg % NB)
        fire_gather(jg, jg % NB)
        wait_gather(b, b)
        fire_scatter(b, b)

    @pl.loop(NB, NCH - NB, step=NB)
    def _(j0):
        for b in range(NB):
            j = j0 + b
            jg = j + PG
            bg = (b + PG) % NB
            wait_scatter(jg - NB, bg)
            fire_gather(jg, bg)
            wait_gather(j, b)
            fire_scatter(j, b)

    for b in range(NB):
        j = NCH - NB + b
        jg = j + PG
        if jg < NCH:
            wait_scatter(jg - NB, jg % NB)
            fire_gather(jg, jg % NB)
        wait_gather(j, b)
        fire_scatter(j, b)
    for b in range(NB):
        wait_scatter(NCH - NB + b, b)

    plsc.subcore_barrier()
    pltpu.sync_copy(acc.at[pl.ds(s * ZR, ZR)],
                    out_hbm.at[c, pl.ds(s * ZR, ZR)])
    if cnt is not None:
        pltpu.sync_copy(acc16.at[pl.ds(s * ZR, ZR)],
                        cnt_hbm.at[c, pl.ds(s * ZR, ZR)])

    @pl.when(s == NS - 1)
    def _():
        pltpu.sync_copy(acc.at[pl.ds(NS * ZR, ZTAIL)],
                        out_hbm.at[c, pl.ds(NS * ZR, ZTAIL)])
        if cnt is not None:
            pltpu.sync_copy(acc16.at[pl.ds(NS * ZR, ZTAIL)],
                            cnt_hbm.at[c, pl.ds(NS * ZR, ZTAIL)])


def _sc_first_body(p_hbm, srcr, dstr, zeros32, zeros16, ones16,
                   out_hbm, cnt_hbm,
                   acc, acc16, src_all, dst_all, rows, ones_v,
                   gsem, ssem, osem):
    _sc_agg_common(p_hbm, srcr, dstr, zeros32, out_hbm,
                   acc, src_all, dst_all, rows, gsem, ssem,
                   cnt=(zeros16, ones16, cnt_hbm, acc16, ones_v, osem))


def _sc_rest_body(p_hbm, srcr, dstr, zeros32, out_hbm,
                  acc, src_all, dst_all, rows, gsem, ssem):
    _sc_agg_common(p_hbm, srcr, dstr, zeros32, out_hbm,
                   acc, src_all, dst_all, rows, gsem, ssem)


_sc_agg_first = pl.kernel(
    _sc_first_body,
    out_type=(jax.ShapeDtypeStruct((NC, NP, HH), jnp.float32),
              jax.ShapeDtypeStruct((NC, NP, 16), jnp.float32)),
    mesh=_MESH,
    compiler_params=pltpu.CompilerParams(use_tc_tiling_on_sc=False),
    scratch_types=[
        pltpu.VMEM_SHARED((NP, HH), jnp.float32),
        pltpu.VMEM_SHARED((NP, 16), jnp.float32),
        pltpu.VMEM((NCH, CH), jnp.int32),
        pltpu.VMEM((NCH, CH), jnp.int32),
        pltpu.VMEM((NB, CH, HH), jnp.float32),
        pltpu.VMEM((CH, 16), jnp.float32),
        pltpu.SemaphoreType.DMA((NB,)),
        pltpu.SemaphoreType.DMA((NB,)),
        pltpu.SemaphoreType.DMA((NB,)),
    ],
)

_sc_agg_rest = pl.kernel(
    _sc_rest_body,
    out_type=jax.ShapeDtypeStruct((NC, NP, HH), jnp.float32),
    mesh=_MESH,
    compiler_params=pltpu.CompilerParams(use_tc_tiling_on_sc=False),
    scratch_types=[
        pltpu.VMEM_SHARED((NP, HH), jnp.float32),
        pltpu.VMEM((NCH, CH), jnp.int32),
        pltpu.VMEM((NCH, CH), jnp.int32),
        pltpu.VMEM((NB, CH, HH), jnp.float32),
        pltpu.SemaphoreType.DMA((NB,)),
        pltpu.SemaphoreType.DMA((NB,)),
    ],
)


def _proj0_body(x_ref, a_ref, b_ref, p_ref, q_ref):
    x = x_ref[...]
    p_ref[...] = lax.dot_general(x, a_ref[...], _DIMS,
                                 preferred_element_type=jnp.float32)
    q_ref[...] = lax.dot_general(x, b_ref[...], _DIMS,
                                 preferred_element_type=jnp.float32)


def _softmax1(v):
    m = jnp.max(v, axis=1, keepdims=True)
    e = jnp.exp(v - m)
    return e / jnp.sum(e, axis=1, keepdims=True)


def _softmax0(v):
    m = jnp.max(v, axis=0, keepdims=True)
    e = jnp.exp(v - m)
    return e / jnp.sum(e, axis=0, keepdims=True)


def _combine(aggp_ref, q_ref, invc_ref):
    agg = aggp_ref[0, 0:NN, :] + aggp_ref[1, 0:NN, :]
    return agg * invc_ref[...] + q_ref[...]


def _round_body(aggp_ref, q_ref, invc_ref, a_ref, b_ref, p_out, q_out, *,
                do_sm0):
    h = _softmax1(_combine(aggp_ref, q_ref, invc_ref))
    if do_sm0:
        h = _softmax0(h)
    p_out[...] = lax.dot_general(h, a_ref[...], _DIMS,
                                 preferred_element_type=jnp.float32)
    q_out[...] = lax.dot_general(h, b_ref[...], _DIMS,
                                 preferred_element_type=jnp.float32)


def _round0_body(aggp_ref, q_ref, cntp_ref, a_ref, b_ref,
                 p_out, q_out, invc_out):
    cnt = cntp_ref[0, 0:NN, 0:1] + cntp_ref[1, 0:NN, 0:1]
    invc_out[...] = jnp.broadcast_to(1.0 / jnp.maximum(cnt, 1.0), (NN, HH))
    h = _softmax1(_combine(aggp_ref, q_ref, invc_out))
    p_out[...] = lax.dot_general(h, a_ref[...], _DIMS,
                                 preferred_element_type=jnp.float32)
    q_out[...] = lax.dot_general(h, b_ref[...], _DIMS,
                                 preferred_element_type=jnp.float32)


def _final_body(aggp_ref, q_ref, invc_ref, w_ref, bo_ref, y_ref):
    h = _softmax0(_softmax1(_combine(aggp_ref, q_ref, invc_ref)))
    z = jnp.sum(h * w_ref[...], axis=1, keepdims=True) + bo_ref[0, 0]
    y_ref[...] = _softmax0(z)


_NH = jax.ShapeDtypeStruct((NN, HH), jnp.float32)

_proj0 = pl.pallas_call(
    _proj0_body, out_shape=(_NH, _NH))

_round0 = pl.pallas_call(
    _round0_body, out_shape=(_NH, _NH, _NH))
_round_sm = pl.pallas_call(
    functools.partial(_round_body, do_sm0=True), out_shape=(_NH, _NH))

_final = pl.pallas_call(
    _final_body, out_shape=jax.ShapeDtypeStruct((NN, 1), jnp.float32))


def kernel(x, edge_index, A0, B0, As, Bs, Wout, bout):
    src2 = edge_index[0].reshape(NW, EPW)
    dst2 = edge_index[1].reshape(NW, EPW)
    if EPAD:
        src2 = jnp.concatenate(
            [src2, jnp.zeros((NW, EPAD), jnp.int32)], axis=1)
        dst2 = jnp.concatenate(
            [dst2, jnp.full((NW, EPAD), NN, jnp.int32)], axis=1)
    srcr = src2.reshape(NW, NCH, CH)
    dstr = dst2.reshape(NW, NCH, CH)
    z32 = jnp.zeros((NP, HH), jnp.float32)
    z16 = jnp.zeros((NP, 16), jnp.float32)
    o16 = jnp.ones((CH, 16), jnp.float32)

    p, q = _proj0(x, A0, B0)
    aggp, cntp = _sc_agg_first(p, srcr, dstr, z32, z16, o16)
    p, q, invc = _round0(aggp, q, cntp, As[0], Bs[0])

    for r in range(1, 16):
        aggp = _sc_agg_rest(p, srcr, dstr, z32)
        p, q = _round_sm(aggp, q, invc, As[r], Bs[r])

    aggp = _sc_agg_rest(p, srcr, dstr, z32)
    return _final(aggp, q, invc, Wout, bout.reshape(1, 1))
